# baseline (speedup 1.0000x reference)
import functools

import jax
import jax.numpy as jnp
from jax import lax
from jax.experimental import pallas as pl
from jax.experimental.pallas import tpu as pltpu
from jax.experimental.pallas import tpu_sc as plsc

D_MODEL = 1024
N_BATCH = 4
SEQ = 4096
N_TOK = N_BATCH * SEQ
N_WORKERS = 32
PER_W = N_TOK // N_WORKERS
CHUNK = 16
N_CHUNKS = PER_W // CHUNK
LANES = 16


@functools.partial(
    pl.kernel,
    mesh=plsc.VectorSubcoreMesh(core_axis_name="c", subcore_axis_name="s"),
    out_type=jax.ShapeDtypeStruct((N_TOK, D_MODEL), jnp.float32),
    scratch_types=[
        pltpu.VMEM((PER_W,), jnp.int32),
        pltpu.VMEM((CHUNK, D_MODEL), jnp.float32),
        pltpu.VMEM((CHUNK, D_MODEL), jnp.float32),
        pltpu.SemaphoreType.DMA,
        pltpu.SemaphoreType.DMA,
    ],
)
def _embed_sc(x_hbm, table_hbm, pos_hbm, out_hbm, idx_v, rows_v, pos_v,
              g_sem, p_sem):
    wid = lax.axis_index("s") * 2 + lax.axis_index("c")
    base = wid * PER_W
    s0 = lax.rem(base, SEQ)
    pltpu.sync_copy(x_hbm.at[pl.ds(base, PER_W)], idx_v)

    def chunk_body(c, carry):
        off = c * CHUNK
        gather = pltpu.async_copy(
            table_hbm.at[idx_v.at[pl.ds(off, CHUNK)]], rows_v, g_sem)
        pos_cp = pltpu.async_copy(
            pos_hbm.at[pl.ds(s0 + off, CHUNK)], pos_v, p_sem)
        gather.wait()
        pos_cp.wait()

        def row_body(r, rcarry):
            for j in range(D_MODEL // LANES):
                sl = pl.ds(j * LANES, LANES)
                rows_v[r, sl] = rows_v[r, sl] + pos_v[r, sl]
            return rcarry
        lax.fori_loop(0, CHUNK, row_body, 0)

        pltpu.sync_copy(rows_v, out_hbm.at[pl.ds(base + off, CHUNK)])
        return carry

    lax.fori_loop(0, N_CHUNKS, chunk_body, 0)


def kernel(x, table, pos_encoding):
    flat_ids = x.reshape(-1).astype(jnp.int32)
    out = _embed_sc(flat_ids, table, pos_encoding)
    return out.reshape(N_BATCH, SEQ, D_MODEL)


# speedup vs baseline: 1.0520x; 1.0520x over previous
import functools

import jax
import jax.numpy as jnp
from jax import lax
from jax.experimental import pallas as pl
from jax.experimental.pallas import tpu as pltpu
from jax.experimental.pallas import tpu_sc as plsc

D_MODEL = 1024
N_BATCH = 4
SEQ = 4096
N_TOK = N_BATCH * SEQ
N_WORKERS = 32
PER_W = N_TOK // N_WORKERS
CHUNK = 16
N_CHUNKS = PER_W // CHUNK
LANES = 16


@functools.partial(
    pl.kernel,
    mesh=plsc.VectorSubcoreMesh(core_axis_name="c", subcore_axis_name="s"),
    out_type=jax.ShapeDtypeStruct((N_TOK, D_MODEL), jnp.float32),
    scratch_types=[
        pltpu.VMEM((PER_W,), jnp.int32),
        pltpu.VMEM((CHUNK, D_MODEL), jnp.float32),
        pltpu.VMEM((CHUNK, D_MODEL), jnp.float32),
        pltpu.SemaphoreType.DMA,
        pltpu.SemaphoreType.DMA,
    ],
)
def _embed_sc(x_hbm, table_hbm, pos_hbm, out_hbm, idx_v, rows_v, pos_v,
              g_sem, p_sem):
    wid = lax.axis_index("s") * 2 + lax.axis_index("c")
    base = wid * PER_W
    s0 = lax.rem(base, SEQ)
    pltpu.sync_copy(x_hbm.at[pl.ds(base, PER_W)], idx_v)

    def chunk_body(c, carry):
        off = c * CHUNK
        gather = pltpu.async_copy(
            table_hbm.at[idx_v.at[pl.ds(off, CHUNK)]], rows_v, g_sem)
        pos_cp = pltpu.async_copy(
            pos_hbm.at[pl.ds(s0 + off, CHUNK)], pos_v, p_sem)
        gather.wait()
        pos_cp.wait()

        def row_body(r, rcarry):
            for j in range(D_MODEL // LANES):
                sl = pl.ds(j * LANES, LANES)
                plsc.addupdate(pos_v.at[r, sl], rows_v[r, sl])
            return rcarry
        lax.fori_loop(0, CHUNK, row_body, 0)

        pltpu.sync_copy(pos_v, out_hbm.at[pl.ds(base + off, CHUNK)])
        return carry

    lax.fori_loop(0, N_CHUNKS, chunk_body, 0)


def kernel(x, table, pos_encoding):
    flat_ids = x.reshape(-1).astype(jnp.int32)
    out = _embed_sc(flat_ids, table, pos_encoding)
    return out.reshape(N_BATCH, SEQ, D_MODEL)


# speedup vs baseline: 1.4575x; 1.3854x over previous
import functools

import jax
import jax.numpy as jnp
from jax import lax
from jax.experimental import pallas as pl
from jax.experimental.pallas import tpu as pltpu
from jax.experimental.pallas import tpu_sc as plsc

D_MODEL = 1024
N_BATCH = 4
SEQ = 4096
N_TOK = N_BATCH * SEQ
N_WORKERS = 32
S_PER_W = SEQ // N_WORKERS
CS = 4
ROWS = N_BATCH * CS
N_CHUNKS = S_PER_W // CS
NB = 4
LANES = 16


@functools.partial(
    pl.kernel,
    mesh=plsc.VectorSubcoreMesh(core_axis_name="c", subcore_axis_name="s"),
    out_type=jax.ShapeDtypeStruct((N_TOK, D_MODEL), jnp.float32),
    scratch_types=(
        [pltpu.VMEM((N_CHUNKS * ROWS,), jnp.int32)]
        + [pltpu.VMEM((ROWS, D_MODEL), jnp.float32) for _ in range(NB)]
        + [pltpu.VMEM((CS, D_MODEL), jnp.float32) for _ in range(NB)]
        + [pltpu.VMEM((ROWS, D_MODEL), jnp.float32) for _ in range(2)]
        + [pltpu.SemaphoreType.DMA for _ in range(2 * NB + 2)]
    ),
)
def _embed_sc(x_hbm, table_hbm, pos_hbm, out_hbm,
              idx_v,
              rows0, rows1, rows2, rows3,
              pos0, pos1, pos2, pos3,
              sb0, sb1,
              g0, g1, g2, g3, p0, p1, p2, p3, o0, o1):
    rows_b = (rows0, rows1, rows2, rows3)
    pos_b = (pos0, pos1, pos2, pos3)
    g_sem = (g0, g1, g2, g3)
    p_sem = (p0, p1, p2, p3)
    sb = (sb0, sb1)
    o_sem = (o0, o1)

    wid = lax.axis_index("s") * 2 + lax.axis_index("c")
    s_base = wid * S_PER_W

    pltpu.sync_copy(x_hbm.at[pl.ds(wid * N_CHUNKS * ROWS, N_CHUNKS * ROWS)],
                    idx_v)

    def issue_chunk(c, slot):
        pltpu.async_copy(table_hbm.at[idx_v.at[pl.ds(c * ROWS, ROWS)]], rows_b[slot], g_sem[slot])
        pltpu.async_copy(pos_hbm.at[pl.ds(s_base + c * CS, CS)],
                         pos_b[slot], p_sem[slot])

    for c in range(NB):
        issue_chunk(c, c)

    def outer(i, carry):
        for b in range(NB):
            c = i * NB + b
            sbi = b % 2
            @pl.when(c >= 2)
            def _():
                for bi in range(N_BATCH):
                    pltpu.make_async_copy(
                        sb[sbi].at[pl.ds(bi * CS, CS)],
                        out_hbm.at[pl.ds(bi * SEQ, CS)],
                        o_sem[sbi]).wait()
            pltpu.make_async_copy(table_hbm.at[idx_v.at[pl.ds(c * ROWS, ROWS)]], rows_b[b],
                                  g_sem[b]).wait()
            pltpu.make_async_copy(pos_hbm.at[pl.ds(0, CS)], pos_b[b],
                                  p_sem[b]).wait()

            def add_body(j, jcarry):
                sl = pl.ds(j * LANES, LANES)
                for s in range(CS):
                    pv = pos_b[b][s, sl]
                    for bi in range(N_BATCH):
                        r = bi * CS + s
                        sb[sbi][r, sl] = rows_b[b][r, sl] + pv
                return jcarry
            lax.fori_loop(0, D_MODEL // LANES, add_body, 0)

            out_row = s_base + c * CS
            for bi in range(N_BATCH):
                pltpu.async_copy(sb[sbi].at[pl.ds(bi * CS, CS)],
                                 out_hbm.at[pl.ds(bi * SEQ + out_row, CS)],
                                 o_sem[sbi])

            @pl.when(c + NB < N_CHUNKS)
            def _():
                issue_chunk(c + NB, b)
        return carry

    lax.fori_loop(0, N_CHUNKS // NB, outer, 0)

    for sbi in range(2):
        for bi in range(N_BATCH):
            pltpu.make_async_copy(
                sb[sbi].at[pl.ds(bi * CS, CS)],
                out_hbm.at[pl.ds(bi * SEQ, CS)],
                o_sem[sbi]).wait()


def kernel(x, table, pos_encoding):
    x_perm = (x.astype(jnp.int32)
              .reshape(N_BATCH, N_WORKERS, N_CHUNKS, CS)
              .transpose(1, 2, 0, 3)
              .reshape(-1))
    out = _embed_sc(x_perm, table, pos_encoding)
    return out.reshape(N_BATCH, SEQ, D_MODEL)


# speedup vs baseline: 1.4860x; 1.0195x over previous
import functools

import jax
import jax.numpy as jnp
from jax import lax
from jax.experimental import pallas as pl
from jax.experimental.pallas import tpu as pltpu
from jax.experimental.pallas import tpu_sc as plsc

D_MODEL = 1024
N_BATCH = 4
SEQ = 4096
N_TOK = N_BATCH * SEQ
N_WORKERS = 32
S_PER_W = SEQ // N_WORKERS
CS = 4
ROWS = N_BATCH * CS
N_CHUNKS = S_PER_W // CS
NB = 4
LANES = 16


@functools.partial(
    pl.kernel,
    mesh=plsc.VectorSubcoreMesh(core_axis_name="c", subcore_axis_name="s"),
    out_type=jax.ShapeDtypeStruct((N_TOK, D_MODEL), jnp.float32),
    scratch_types=(
        [pltpu.VMEM((N_BATCH * S_PER_W,), jnp.int32)]
        + [pltpu.VMEM((ROWS, D_MODEL), jnp.float32) for _ in range(NB)]
        + [pltpu.VMEM((CS, D_MODEL), jnp.float32) for _ in range(NB)]
        + [pltpu.VMEM((ROWS, D_MODEL), jnp.float32) for _ in range(2)]
        + [pltpu.SemaphoreType.DMA for _ in range(2 * NB + 2)]
    ),
    compiler_params=pltpu.CompilerParams(needs_layout_passes=False),
)
def _embed_sc(x_hbm, table_hbm, pos_hbm, out_hbm,
              stage_v,
              rows0, rows1, rows2, rows3,
              pos0, pos1, pos2, pos3,
              sb0, sb1,
              g0, g1, g2, g3, p0, p1, p2, p3, o0, o1):
    rows_b = (rows0, rows1, rows2, rows3)
    pos_b = (pos0, pos1, pos2, pos3)
    g_sem = (g0, g1, g2, g3)
    p_sem = (p0, p1, p2, p3)
    sb = (sb0, sb1)
    o_sem = (o0, o1)

    wid = lax.axis_index("s") * 2 + lax.axis_index("c")
    s_base = wid * S_PER_W

    for bi in range(N_BATCH):
        pltpu.sync_copy(x_hbm.at[pl.ds(bi * SEQ + s_base, S_PER_W)],
                        stage_v.at[pl.ds(bi * S_PER_W, S_PER_W)])

    io = lax.iota(jnp.int32, LANES)
    lane_off = lax.shift_right_logical(io, 2) * S_PER_W + jnp.bitwise_and(io, 3)

    def issue_chunk(c, slot):
        ids = plsc.load_gather(stage_v, [lane_off + c * CS])
        pltpu.async_copy(table_hbm.at[ids], rows_b[slot], g_sem[slot])
        pltpu.async_copy(pos_hbm.at[pl.ds(s_base + c * CS, CS)],
                         pos_b[slot], p_sem[slot])

    for c in range(NB):
        issue_chunk(c, c)

    def outer(i, carry):
        for b in range(NB):
            c = i * NB + b
            sbi = b % 2
            @pl.when(c >= 2)
            def _():
                for bi in range(N_BATCH):
                    pltpu.make_async_copy(
                        sb[sbi].at[pl.ds(bi * CS, CS)],
                        out_hbm.at[pl.ds(bi * SEQ, CS)],
                        o_sem[sbi]).wait()
            pltpu.make_async_copy(table_hbm.at[io], rows_b[b],
                                  g_sem[b]).wait()
            pltpu.make_async_copy(pos_hbm.at[pl.ds(0, CS)], pos_b[b],
                                  p_sem[b]).wait()

            def add_body(j, jcarry):
                sl = pl.ds(j * LANES, LANES)
                for s in range(CS):
                    pv = pos_b[b][s, sl]
                    for bi in range(N_BATCH):
                        r = bi * CS + s
                        sb[sbi][r, sl] = rows_b[b][r, sl] + pv
                return jcarry
            lax.fori_loop(0, D_MODEL // LANES, add_body, 0)

            out_row = s_base + c * CS
            for bi in range(N_BATCH):
                pltpu.async_copy(sb[sbi].at[pl.ds(bi * CS, CS)],
                                 out_hbm.at[pl.ds(bi * SEQ + out_row, CS)],
                                 o_sem[sbi])

            @pl.when(c + NB < N_CHUNKS)
            def _():
                issue_chunk(c + NB, b)
        return carry

    lax.fori_loop(0, N_CHUNKS // NB, outer, 0)

    for sbi in range(2):
        for bi in range(N_BATCH):
            pltpu.make_async_copy(
                sb[sbi].at[pl.ds(bi * CS, CS)],
                out_hbm.at[pl.ds(bi * SEQ, CS)],
                o_sem[sbi]).wait()


def kernel(x, table, pos_encoding):
    out = _embed_sc(x.reshape(-1).astype(jnp.int32), table, pos_encoding)
    return out.reshape(N_BATCH, SEQ, D_MODEL)
